# baseline (speedup 1.0000x reference)
import functools

import jax
import jax.numpy as jnp
from jax import lax
from jax.experimental import pallas as pl
from jax.experimental.pallas import tpu as pltpu
from jax.experimental.pallas import tpu_sc as plsc

N_NODES = 10000
D = 128
E = 320000
NC = 2
NS = 16
NW = NC * NS
CHUNK = 128
NCHUNK = 80
EPW = NCHUNK * CHUNK
E_PAD = NW * EPW
N_PAD = 10240
ROWS_PER_TILE = N_PAD // NS
ZROWS = 128


SDEPTH = 40
NSTAGE = NCHUNK // SDEPTH


NBUF = 2


def _hop_body(src_hbm, row_hbm, col_hbm, w_hbm, parts_hbm,
              acc_sh, rows0, rows1, ridx_v, cidx_v, w_v,
              gsem0, gsem1, ssem0, ssem1):
  cid = lax.axis_index("c")
  sid = lax.axis_index("s")
  wid = sid * NC + cid

  bufs = (rows0, rows1)
  gsems = (gsem0, gsem1)
  ssems = (ssem0, ssem1)

  def _zero_row(r, _):
    for j in range(D // 16):
      rows0[r, pl.ds(j * 16, 16)] = jnp.zeros((16,), jnp.float32)
    return 0
  lax.fori_loop(0, ZROWS, _zero_row, 0)
  for k in range(ROWS_PER_TILE // ZROWS):
    pltpu.sync_copy(rows0, acc_sh.at[pl.ds(sid * ROWS_PER_TILE + k * ZROWS,
                                           ZROWS)])
  plsc.subcore_barrier()

  for s in range(NSTAGE):
    cbase = wid * NCHUNK + s * SDEPTH
    pltpu.sync_copy(row_hbm.at[pl.ds(cbase, SDEPTH)], ridx_v)
    pltpu.sync_copy(col_hbm.at[pl.ds(cbase, SDEPTH)], cidx_v)
    pltpu.sync_copy(w_hbm.at[pl.ds(cbase, SDEPTH)], w_v)

    pltpu.async_copy(src_hbm.at[ridx_v.at[0]], bufs[0], gsems[0])
    pltpu.async_copy(src_hbm.at[ridx_v.at[1]], bufs[1], gsems[1])

    def _pair(k, _):
      for b in range(NBUF):
        lj = NBUF * k + b
        ob = 1 - b

        pltpu.make_async_copy(src_hbm.at[ridx_v.at[lj]], bufs[b],
                              gsems[b]).wait()

        def _scale(e, _):
          ws = plsc.load_gather(w_v, [jnp.full((16,), lj, jnp.int32),
                                      jnp.full((16,), e, jnp.int32)])
          for j in range(D // 16):
            sl = pl.ds(j * 16, 16)
            bufs[b][e, sl] = bufs[b][e, sl] * ws
          return 0
        lax.fori_loop(0, CHUNK, _scale, 0)

        pltpu.async_copy(bufs[b], acc_sh.at[cidx_v.at[lj]], ssems[b],
                         add=True)

        @pl.when(jnp.logical_and(lj >= 1, lj + 1 < SDEPTH))
        def _():
          pltpu.make_async_copy(
              bufs[ob], acc_sh.at[cidx_v.at[lj - 1]], ssems[ob]).wait()
          pltpu.async_copy(src_hbm.at[ridx_v.at[lj + 1]], bufs[ob],
                           gsems[ob])
      return 0
    lax.fori_loop(0, SDEPTH // NBUF, _pair, 0)

    for lj in range(SDEPTH - NBUF, SDEPTH):
      pltpu.make_async_copy(
          bufs[lj % NBUF], acc_sh.at[cidx_v.at[lj]], ssems[lj % NBUF]).wait()

  plsc.subcore_barrier()

  for k in range(ROWS_PER_TILE // ZROWS):
    off = sid * ROWS_PER_TILE + k * ZROWS
    pltpu.sync_copy(acc_sh.at[pl.ds(off, ZROWS)], rows0)
    pltpu.sync_copy(rows0, parts_hbm.at[cid, pl.ds(off, ZROWS)])


_hop = pl.kernel(
    _hop_body,
    out_type=jax.ShapeDtypeStruct((NC, N_PAD, D), jnp.float32),
    mesh=plsc.VectorSubcoreMesh(core_axis_name="c", subcore_axis_name="s",
                                num_cores=NC, num_subcores=NS),
    scratch_types=[
        pltpu.VMEM_SHARED((N_PAD, D), jnp.float32),
        pltpu.VMEM((CHUNK, D), jnp.float32),
        pltpu.VMEM((CHUNK, D), jnp.float32),
        pltpu.VMEM((SDEPTH, CHUNK), jnp.int32),
        pltpu.VMEM((SDEPTH, CHUNK), jnp.int32),
        pltpu.VMEM((SDEPTH, CHUNK), jnp.float32),
        pltpu.SemaphoreType.DMA,
        pltpu.SemaphoreType.DMA,
        pltpu.SemaphoreType.DMA,
        pltpu.SemaphoreType.DMA,
    ],
    compiler_params=pltpu.CompilerParams(needs_layout_passes=False),
)


def _add_body(a_ref, b_ref, o_ref):
  o_ref[...] = a_ref[...] + b_ref[...]


_BLK = 2000


def _combine(parts):
  return pl.pallas_call(
      _add_body,
      grid=(N_NODES // _BLK,),
      in_specs=[pl.BlockSpec((_BLK, D), lambda i: (i, 0)),
                pl.BlockSpec((_BLK, D), lambda i: (i, 0))],
      out_specs=pl.BlockSpec((_BLK, D), lambda i: (i, 0)),
      out_shape=jax.ShapeDtypeStruct((N_NODES, D), jnp.float32),
  )(parts[0, :N_NODES], parts[1, :N_NODES])


@jax.jit
def kernel(embed, adj_sp_norm, edge_index, edge_weight, deg):
  pad = E_PAD - E
  row = jnp.concatenate([edge_index[0], jnp.zeros((pad,), jnp.int32)])
  col = jnp.concatenate([edge_index[1], jnp.zeros((pad,), jnp.int32)])
  w = jnp.concatenate([edge_weight, jnp.zeros((pad,), jnp.float32)])
  row2d = row.reshape(NW * NCHUNK, CHUNK)
  col2d = col.reshape(NW * NCHUNK, CHUNK)
  w2d = w.reshape(NW * NCHUNK, CHUNK)

  embs = [embed]
  a = embed
  for _ in range(3):
    parts = _hop(a, row2d, col2d, w2d)
    a = _combine(parts)
    embs.append(a)
  embs = jnp.stack(embs, axis=1)
  return (embs[: N_NODES // 2], embs[N_NODES // 2:])


# speedup vs baseline: 1.0842x; 1.0842x over previous
import functools

import jax
import jax.numpy as jnp
from jax import lax
from jax.experimental import pallas as pl
from jax.experimental.pallas import tpu as pltpu
from jax.experimental.pallas import tpu_sc as plsc

N_NODES = 10000
D = 128
E = 320000
NC = 2
NS = 16
NW = NC * NS
CHUNK = 128
NCHUNK = 80
EPW = NCHUNK * CHUNK
E_PAD = NW * EPW
N_PAD = 10240
ROWS_PER_TILE = N_PAD // NS
ZROWS = 128


SDEPTH = 40
NSTAGE = NCHUNK // SDEPTH


NBUF = 2


def _hop_body(src_hbm, row_hbm, col_hbm, w_hbm, parts_hbm,
              acc_sh, rows0, rows1, ridx_v, cidx_v, w_v,
              gsem0, gsem1, ssem0, ssem1):
  cid = lax.axis_index("c")
  sid = lax.axis_index("s")
  wid = sid * NC + cid

  bufs = (rows0, rows1)
  gsems = (gsem0, gsem1)
  ssems = (ssem0, ssem1)

  def _zero_row(r, _):
    for j in range(D // 16):
      rows0[r, pl.ds(j * 16, 16)] = jnp.zeros((16,), jnp.float32)
    return 0
  lax.fori_loop(0, ZROWS, _zero_row, 0)
  for k in range(ROWS_PER_TILE // ZROWS):
    pltpu.sync_copy(rows0, acc_sh.at[pl.ds(sid * ROWS_PER_TILE + k * ZROWS,
                                           ZROWS)])
  plsc.subcore_barrier()

  for s in range(NSTAGE):
    cbase = wid * NCHUNK + s * SDEPTH
    pltpu.sync_copy(row_hbm.at[pl.ds(cbase, SDEPTH)], ridx_v)
    pltpu.sync_copy(col_hbm.at[pl.ds(cbase, SDEPTH)], cidx_v)
    pltpu.sync_copy(w_hbm.at[pl.ds(cbase, SDEPTH)], w_v)

    pltpu.async_copy(src_hbm.at[ridx_v.at[0]], bufs[0], gsems[0])
    pltpu.async_copy(src_hbm.at[ridx_v.at[1]], bufs[1], gsems[1])

    def _pair(k, _):
      for b in range(NBUF):
        lj = NBUF * k + b
        ob = 1 - b

        pltpu.make_async_copy(src_hbm.at[ridx_v.at[lj]], bufs[b],
                              gsems[b]).wait()

        @plsc.parallel_loop(0, CHUNK, 1, unroll=4)
        def _scale(e):
          ws = plsc.load_gather(w_v, [jnp.full((16,), lj, jnp.int32),
                                      jnp.full((16,), e, jnp.int32)])
          for j in range(D // 16):
            sl = pl.ds(j * 16, 16)
            bufs[b][e, sl] = bufs[b][e, sl] * ws

        pltpu.async_copy(bufs[b], acc_sh.at[cidx_v.at[lj]], ssems[b],
                         add=True)

        @pl.when(jnp.logical_and(lj >= 1, lj + 1 < SDEPTH))
        def _():
          pltpu.make_async_copy(
              bufs[ob], acc_sh.at[cidx_v.at[lj - 1]], ssems[ob]).wait()
          pltpu.async_copy(src_hbm.at[ridx_v.at[lj + 1]], bufs[ob],
                           gsems[ob])
      return 0
    lax.fori_loop(0, SDEPTH // NBUF, _pair, 0)

    for lj in range(SDEPTH - NBUF, SDEPTH):
      pltpu.make_async_copy(
          bufs[lj % NBUF], acc_sh.at[cidx_v.at[lj]], ssems[lj % NBUF]).wait()

  plsc.subcore_barrier()

  for k in range(ROWS_PER_TILE // ZROWS):
    off = sid * ROWS_PER_TILE + k * ZROWS
    pltpu.sync_copy(acc_sh.at[pl.ds(off, ZROWS)], rows0)
    pltpu.sync_copy(rows0, parts_hbm.at[cid, pl.ds(off, ZROWS)])


_hop = pl.kernel(
    _hop_body,
    out_type=jax.ShapeDtypeStruct((NC, N_PAD, D), jnp.float32),
    mesh=plsc.VectorSubcoreMesh(core_axis_name="c", subcore_axis_name="s",
                                num_cores=NC, num_subcores=NS),
    scratch_types=[
        pltpu.VMEM_SHARED((N_PAD, D), jnp.float32),
        pltpu.VMEM((CHUNK, D), jnp.float32),
        pltpu.VMEM((CHUNK, D), jnp.float32),
        pltpu.VMEM((SDEPTH, CHUNK), jnp.int32),
        pltpu.VMEM((SDEPTH, CHUNK), jnp.int32),
        pltpu.VMEM((SDEPTH, CHUNK), jnp.float32),
        pltpu.SemaphoreType.DMA,
        pltpu.SemaphoreType.DMA,
        pltpu.SemaphoreType.DMA,
        pltpu.SemaphoreType.DMA,
    ],
    compiler_params=pltpu.CompilerParams(needs_layout_passes=False),
)


def _add_body(a_ref, b_ref, o_ref):
  o_ref[...] = a_ref[...] + b_ref[...]


_BLK = 2000


def _combine(parts):
  return pl.pallas_call(
      _add_body,
      grid=(N_NODES // _BLK,),
      in_specs=[pl.BlockSpec((_BLK, D), lambda i: (i, 0)),
                pl.BlockSpec((_BLK, D), lambda i: (i, 0))],
      out_specs=pl.BlockSpec((_BLK, D), lambda i: (i, 0)),
      out_shape=jax.ShapeDtypeStruct((N_NODES, D), jnp.float32),
  )(parts[0, :N_NODES], parts[1, :N_NODES])


@jax.jit
def kernel(embed, adj_sp_norm, edge_index, edge_weight, deg):
  pad = E_PAD - E
  row = jnp.concatenate([edge_index[0], jnp.zeros((pad,), jnp.int32)])
  col = jnp.concatenate([edge_index[1], jnp.zeros((pad,), jnp.int32)])
  w = jnp.concatenate([edge_weight, jnp.zeros((pad,), jnp.float32)])
  row2d = row.reshape(NW * NCHUNK, CHUNK)
  col2d = col.reshape(NW * NCHUNK, CHUNK)
  w2d = w.reshape(NW * NCHUNK, CHUNK)

  embs = [embed]
  a = embed
  for _ in range(3):
    parts = _hop(a, row2d, col2d, w2d)
    a = _combine(parts)
    embs.append(a)
  embs = jnp.stack(embs, axis=1)
  return (embs[: N_NODES // 2], embs[N_NODES // 2:])


# speedup vs baseline: 1.2009x; 1.1076x over previous
import functools

import jax
import jax.numpy as jnp
from jax import lax
from jax.experimental import pallas as pl
from jax.experimental.pallas import tpu as pltpu
from jax.experimental.pallas import tpu_sc as plsc

N_NODES = 10000
D = 128
E = 320000
NC = 2
NS = 16
NW = NC * NS
CHUNK = 128
NCHUNK = 80
EPW = NCHUNK * CHUNK
E_PAD = NW * EPW
N_PAD = 10240
ROWS_PER_TILE = N_PAD // NS
ZROWS = 128


SDEPTH = 40
NSTAGE = NCHUNK // SDEPTH


NBUF = 2


def _hop_body(src_hbm, row_hbm, col_hbm, w_hbm, parts_hbm,
              acc_sh, rows0, rows1, ridx_v, cidx_v, w_v,
              gsem0, gsem1, ssem0, ssem1):
  cid = lax.axis_index("c")
  sid = lax.axis_index("s")
  wid = sid * NC + cid

  bufs = (rows0, rows1)
  gsems = (gsem0, gsem1)
  ssems = (ssem0, ssem1)

  def _zero_row(r, _):
    for j in range(D // 16):
      rows0[r, pl.ds(j * 16, 16)] = jnp.zeros((16,), jnp.float32)
    return 0
  lax.fori_loop(0, ZROWS, _zero_row, 0)
  for k in range(ROWS_PER_TILE // ZROWS):
    pltpu.sync_copy(rows0, acc_sh.at[pl.ds(sid * ROWS_PER_TILE + k * ZROWS,
                                           ZROWS)])
  plsc.subcore_barrier()

  for s in range(NSTAGE):
    cbase = wid * NCHUNK + s * SDEPTH
    pltpu.sync_copy(row_hbm.at[pl.ds(cbase, SDEPTH)], ridx_v)
    pltpu.sync_copy(col_hbm.at[pl.ds(cbase, SDEPTH)], cidx_v)
    pltpu.sync_copy(w_hbm.at[pl.ds(cbase, SDEPTH)], w_v)

    pltpu.async_copy(src_hbm.at[ridx_v.at[0]], bufs[0], gsems[0])
    pltpu.async_copy(src_hbm.at[ridx_v.at[1]], bufs[1], gsems[1])

    def _pair(k, _):
      for b in range(NBUF):
        lj = NBUF * k + b

        pltpu.make_async_copy(src_hbm.at[ridx_v.at[lj]], bufs[b],
                              gsems[b]).wait()

        @plsc.parallel_loop(0, CHUNK, 1, unroll=4)
        def _scale(e):
          ws = plsc.load_gather(w_v, [jnp.full((16,), lj, jnp.int32),
                                      jnp.full((16,), e, jnp.int32)])
          for j in range(D // 16):
            sl = pl.ds(j * 16, 16)
            bufs[b][e, sl] = bufs[b][e, sl] * ws

        pltpu.sync_copy(bufs[b], acc_sh.at[cidx_v.at[lj]], add=True)

        @pl.when(lj + 2 < SDEPTH)
        def _():
          pltpu.async_copy(src_hbm.at[ridx_v.at[lj + 2]], bufs[b], gsems[b])
      return 0
    lax.fori_loop(0, SDEPTH // NBUF, _pair, 0)

  plsc.subcore_barrier()

  for k in range(ROWS_PER_TILE // ZROWS):
    off = sid * ROWS_PER_TILE + k * ZROWS
    pltpu.sync_copy(acc_sh.at[pl.ds(off, ZROWS)], rows0)
    pltpu.sync_copy(rows0, parts_hbm.at[cid, pl.ds(off, ZROWS)])


_hop = pl.kernel(
    _hop_body,
    out_type=jax.ShapeDtypeStruct((NC, N_PAD, D), jnp.float32),
    mesh=plsc.VectorSubcoreMesh(core_axis_name="c", subcore_axis_name="s",
                                num_cores=NC, num_subcores=NS),
    scratch_types=[
        pltpu.VMEM_SHARED((N_PAD, D), jnp.float32),
        pltpu.VMEM((CHUNK, D), jnp.float32),
        pltpu.VMEM((CHUNK, D), jnp.float32),
        pltpu.VMEM((SDEPTH, CHUNK), jnp.int32),
        pltpu.VMEM((SDEPTH, CHUNK), jnp.int32),
        pltpu.VMEM((SDEPTH, CHUNK), jnp.float32),
        pltpu.SemaphoreType.DMA,
        pltpu.SemaphoreType.DMA,
        pltpu.SemaphoreType.DMA,
        pltpu.SemaphoreType.DMA,
    ],
    compiler_params=pltpu.CompilerParams(needs_layout_passes=False),
)


def _add_body(a_ref, b_ref, o_ref):
  o_ref[...] = a_ref[...] + b_ref[...]


_BLK = 2000


def _combine(parts):
  return pl.pallas_call(
      _add_body,
      grid=(N_NODES // _BLK,),
      in_specs=[pl.BlockSpec((_BLK, D), lambda i: (i, 0)),
                pl.BlockSpec((_BLK, D), lambda i: (i, 0))],
      out_specs=pl.BlockSpec((_BLK, D), lambda i: (i, 0)),
      out_shape=jax.ShapeDtypeStruct((N_NODES, D), jnp.float32),
  )(parts[0, :N_NODES], parts[1, :N_NODES])


@jax.jit
def kernel(embed, adj_sp_norm, edge_index, edge_weight, deg):
  pad = E_PAD - E
  row = jnp.concatenate([edge_index[0], jnp.zeros((pad,), jnp.int32)])
  col = jnp.concatenate([edge_index[1], jnp.zeros((pad,), jnp.int32)])
  w = jnp.concatenate([edge_weight, jnp.zeros((pad,), jnp.float32)])
  row2d = row.reshape(NW * NCHUNK, CHUNK)
  col2d = col.reshape(NW * NCHUNK, CHUNK)
  w2d = w.reshape(NW * NCHUNK, CHUNK)

  embs = [embed]
  a = embed
  for _ in range(3):
    parts = _hop(a, row2d, col2d, w2d)
    a = _combine(parts)
    embs.append(a)
  embs = jnp.stack(embs, axis=1)
  return (embs[: N_NODES // 2], embs[N_NODES // 2:])


# speedup vs baseline: 1.2956x; 1.0789x over previous
import functools

import jax
import jax.numpy as jnp
from jax import lax
from jax.experimental import pallas as pl
from jax.experimental.pallas import tpu as pltpu
from jax.experimental.pallas import tpu_sc as plsc

N_NODES = 10000
D = 128
E = 320000
NC = 2
NS = 16
NW = NC * NS
CHUNK = 128
K0 = 120
K1 = 40
NCHUNK = K0 + K1
E_PAD = NS * NCHUNK * CHUNK
N_PAD = 10240
ROWS_PER_TILE = N_PAD // NS
ZROWS = 128


SDEPTH = 40
NSTAGE0 = K0 // SDEPTH
NSTAGE1 = K1 // SDEPTH


NBUF = 2


def _hop_body(src_hbm, row_hbm, col_hbm, w_hbm, parts_hbm,
              acc_sh, rows0, rows1, ridx_v, cidx_v, w_v,
              gsem0, gsem1, ssem0, ssem1):
  cid = lax.axis_index("c")
  sid = lax.axis_index("s")
  wid = sid * NC + cid

  bufs = (rows0, rows1)
  gsems = (gsem0, gsem1)
  ssems = (ssem0, ssem1)

  def _zero_row(r, _):
    for j in range(D // 16):
      rows0[r, pl.ds(j * 16, 16)] = jnp.zeros((16,), jnp.float32)
    return 0
  lax.fori_loop(0, ZROWS, _zero_row, 0)
  for k in range(ROWS_PER_TILE // ZROWS):
    pltpu.sync_copy(rows0, acc_sh.at[pl.ds(sid * ROWS_PER_TILE + k * ZROWS,
                                           ZROWS)])
  plsc.subcore_barrier()

  def _run_stage(cbase):
    pltpu.sync_copy(row_hbm.at[pl.ds(cbase, SDEPTH)], ridx_v)
    pltpu.sync_copy(col_hbm.at[pl.ds(cbase, SDEPTH)], cidx_v)
    pltpu.sync_copy(w_hbm.at[pl.ds(cbase, SDEPTH)], w_v)

    pltpu.async_copy(src_hbm.at[ridx_v.at[0]], bufs[0], gsems[0])
    pltpu.async_copy(src_hbm.at[ridx_v.at[1]], bufs[1], gsems[1])

    def _pair(k, _):
      for b in range(NBUF):
        lj = NBUF * k + b

        pltpu.make_async_copy(src_hbm.at[ridx_v.at[lj]], bufs[b],
                              gsems[b]).wait()

        @plsc.parallel_loop(0, CHUNK, 1, unroll=4)
        def _scale(e):
          ws = plsc.load_gather(w_v, [jnp.full((16,), lj, jnp.int32),
                                      jnp.full((16,), e, jnp.int32)])
          for j in range(D // 16):
            sl = pl.ds(j * 16, 16)
            bufs[b][e, sl] = bufs[b][e, sl] * ws

        pltpu.sync_copy(bufs[b], acc_sh.at[cidx_v.at[lj]], add=True)

        @pl.when(lj + 2 < SDEPTH)
        def _():
          pltpu.async_copy(src_hbm.at[ridx_v.at[lj + 2]], bufs[b], gsems[b])
      return 0
    lax.fori_loop(0, SDEPTH // NBUF, _pair, 0)

  for s in range(NSTAGE0):
    if s < NSTAGE1:
      cbase = jnp.where(cid == 0, sid * K0 + s * SDEPTH,
                        NS * K0 + sid * K1 + s * SDEPTH)
      _run_stage(pl.multiple_of(cbase, 8))
    else:
      @pl.when(cid == 0)
      def _():
        _run_stage(pl.multiple_of(sid * K0 + s * SDEPTH, 8))

  plsc.subcore_barrier()

  for k in range(ROWS_PER_TILE // ZROWS):
    off = sid * ROWS_PER_TILE + k * ZROWS
    pltpu.sync_copy(acc_sh.at[pl.ds(off, ZROWS)], rows0)
    pltpu.sync_copy(rows0, parts_hbm.at[cid, pl.ds(off, ZROWS)])


_hop = pl.kernel(
    _hop_body,
    out_type=jax.ShapeDtypeStruct((NC, N_PAD, D), jnp.float32),
    mesh=plsc.VectorSubcoreMesh(core_axis_name="c", subcore_axis_name="s",
                                num_cores=NC, num_subcores=NS),
    scratch_types=[
        pltpu.VMEM_SHARED((N_PAD, D), jnp.float32),
        pltpu.VMEM((CHUNK, D), jnp.float32),
        pltpu.VMEM((CHUNK, D), jnp.float32),
        pltpu.VMEM((SDEPTH, CHUNK), jnp.int32),
        pltpu.VMEM((SDEPTH, CHUNK), jnp.int32),
        pltpu.VMEM((SDEPTH, CHUNK), jnp.float32),
        pltpu.SemaphoreType.DMA,
        pltpu.SemaphoreType.DMA,
        pltpu.SemaphoreType.DMA,
        pltpu.SemaphoreType.DMA,
    ],
    compiler_params=pltpu.CompilerParams(needs_layout_passes=False),
)


def _add_body(a_ref, b_ref, o_ref):
  o_ref[...] = a_ref[...] + b_ref[...]


_BLK = 2000


def _combine(parts):
  return pl.pallas_call(
      _add_body,
      grid=(N_NODES // _BLK,),
      in_specs=[pl.BlockSpec((_BLK, D), lambda i: (i, 0)),
                pl.BlockSpec((_BLK, D), lambda i: (i, 0))],
      out_specs=pl.BlockSpec((_BLK, D), lambda i: (i, 0)),
      out_shape=jax.ShapeDtypeStruct((N_NODES, D), jnp.float32),
  )(parts[0, :N_NODES], parts[1, :N_NODES])


@jax.jit
def kernel(embed, adj_sp_norm, edge_index, edge_weight, deg):
  pad = E_PAD - E
  row = jnp.concatenate([edge_index[0], jnp.zeros((pad,), jnp.int32)])
  col = jnp.concatenate([edge_index[1], jnp.zeros((pad,), jnp.int32)])
  w = jnp.concatenate([edge_weight, jnp.zeros((pad,), jnp.float32)])
  row2d = row.reshape(NS * NCHUNK, CHUNK)
  col2d = col.reshape(NS * NCHUNK, CHUNK)
  w2d = w.reshape(NS * NCHUNK, CHUNK)

  embs = [embed]
  a = embed
  for _ in range(3):
    parts = _hop(a, row2d, col2d, w2d)
    a = _combine(parts)
    embs.append(a)
  embs = jnp.stack(embs, axis=1)
  return (embs[: N_NODES // 2], embs[N_NODES // 2:])


# speedup vs baseline: 1.2989x; 1.0025x over previous
import functools

import jax
import jax.numpy as jnp
from jax import lax
from jax.experimental import pallas as pl
from jax.experimental.pallas import tpu as pltpu
from jax.experimental.pallas import tpu_sc as plsc

N_NODES = 10000
D = 128
E = 320000
NC = 2
NS = 16
NW = NC * NS
CHUNK = 128
K0 = 120
K1 = 40
NCHUNK = K0 + K1
E_PAD = NS * NCHUNK * CHUNK
N_PAD = 10240
ROWS_PER_TILE = N_PAD // NS
ZROWS = 128


SDEPTH = 40
NSTAGE0 = K0 // SDEPTH
NSTAGE1 = K1 // SDEPTH


NBUF = 2


def _hop_body(src_hbm, row_hbm, col_hbm, w_hbm, parts_hbm,
              acc_sh, rows0, rows1, ridx_v, cidx_v, w_v,
              gsem0, gsem1, ssem0, ssem1):
  cid = lax.axis_index("c")
  sid = lax.axis_index("s")
  wid = sid * NC + cid

  bufs = (rows0, rows1)
  gsems = (gsem0, gsem1)
  ssems = (ssem0, ssem1)

  def _zero_row(r, _):
    for j in range(D // 16):
      rows0[r, pl.ds(j * 16, 16)] = jnp.zeros((16,), jnp.float32)
    return 0
  lax.fori_loop(0, ZROWS, _zero_row, 0)
  for k in range(ROWS_PER_TILE // ZROWS):
    pltpu.async_copy(rows0, acc_sh.at[pl.ds(sid * ROWS_PER_TILE + k * ZROWS,
                                            ZROWS)], ssem0)
  for k in range(ROWS_PER_TILE // ZROWS):
    pltpu.make_async_copy(rows0, acc_sh.at[pl.ds(sid * ROWS_PER_TILE
                                                 + k * ZROWS, ZROWS)],
                          ssem0).wait()
  plsc.subcore_barrier()

  def _run_stage(cbase):
    pltpu.async_copy(row_hbm.at[pl.ds(cbase, SDEPTH)], ridx_v, ssem1)
    pltpu.async_copy(col_hbm.at[pl.ds(cbase, SDEPTH)], cidx_v, ssem1)
    pltpu.async_copy(w_hbm.at[pl.ds(cbase, SDEPTH)], w_v, ssem1)
    pltpu.make_async_copy(row_hbm.at[pl.ds(cbase, SDEPTH)], ridx_v,
                          ssem1).wait()
    pltpu.make_async_copy(col_hbm.at[pl.ds(cbase, SDEPTH)], cidx_v,
                          ssem1).wait()
    pltpu.make_async_copy(w_hbm.at[pl.ds(cbase, SDEPTH)], w_v, ssem1).wait()

    pltpu.async_copy(src_hbm.at[ridx_v.at[0]], bufs[0], gsems[0])
    pltpu.async_copy(src_hbm.at[ridx_v.at[1]], bufs[1], gsems[1])

    def _pair(k, _):
      for b in range(NBUF):
        lj = NBUF * k + b

        pltpu.make_async_copy(src_hbm.at[ridx_v.at[lj]], bufs[b],
                              gsems[b]).wait()

        @plsc.parallel_loop(0, CHUNK, 1, unroll=4)
        def _scale(e):
          ws = plsc.load_gather(w_v, [jnp.full((16,), lj, jnp.int32),
                                      jnp.full((16,), e, jnp.int32)])
          for j in range(D // 16):
            sl = pl.ds(j * 16, 16)
            bufs[b][e, sl] = bufs[b][e, sl] * ws

        pltpu.sync_copy(bufs[b], acc_sh.at[cidx_v.at[lj]], add=True)

        @pl.when(lj + 2 < SDEPTH)
        def _():
          pltpu.async_copy(src_hbm.at[ridx_v.at[lj + 2]], bufs[b], gsems[b])
      return 0
    lax.fori_loop(0, SDEPTH // NBUF, _pair, 0)

  for s in range(NSTAGE0):
    if s < NSTAGE1:
      cbase = jnp.where(cid == 0, sid * K0 + s * SDEPTH,
                        NS * K0 + sid * K1 + s * SDEPTH)
      _run_stage(pl.multiple_of(cbase, 8))
    else:
      @pl.when(cid == 0)
      def _():
        _run_stage(pl.multiple_of(sid * K0 + s * SDEPTH, 8))

  plsc.subcore_barrier()

  base = sid * ROWS_PER_TILE
  pltpu.async_copy(acc_sh.at[pl.ds(base, ROWS_PER_TILE)],
                   parts_hbm.at[cid, pl.ds(base, ROWS_PER_TILE)], ssem0)
  pltpu.make_async_copy(acc_sh.at[pl.ds(base, ROWS_PER_TILE)],
                        parts_hbm.at[cid, pl.ds(base, ROWS_PER_TILE)],
                        ssem0).wait()


_hop = pl.kernel(
    _hop_body,
    out_type=jax.ShapeDtypeStruct((NC, N_PAD, D), jnp.float32),
    mesh=plsc.VectorSubcoreMesh(core_axis_name="c", subcore_axis_name="s",
                                num_cores=NC, num_subcores=NS),
    scratch_types=[
        pltpu.VMEM_SHARED((N_PAD, D), jnp.float32),
        pltpu.VMEM((CHUNK, D), jnp.float32),
        pltpu.VMEM((CHUNK, D), jnp.float32),
        pltpu.VMEM((SDEPTH, CHUNK), jnp.int32),
        pltpu.VMEM((SDEPTH, CHUNK), jnp.int32),
        pltpu.VMEM((SDEPTH, CHUNK), jnp.float32),
        pltpu.SemaphoreType.DMA,
        pltpu.SemaphoreType.DMA,
        pltpu.SemaphoreType.DMA,
        pltpu.SemaphoreType.DMA,
    ],
    compiler_params=pltpu.CompilerParams(needs_layout_passes=False),
)


def _add_body(a_ref, b_ref, o_ref):
  o_ref[...] = a_ref[...] + b_ref[...]


_BLK = 2000


def _combine(parts):
  return pl.pallas_call(
      _add_body,
      grid=(N_NODES // _BLK,),
      in_specs=[pl.BlockSpec((_BLK, D), lambda i: (i, 0)),
                pl.BlockSpec((_BLK, D), lambda i: (i, 0))],
      out_specs=pl.BlockSpec((_BLK, D), lambda i: (i, 0)),
      out_shape=jax.ShapeDtypeStruct((N_NODES, D), jnp.float32),
  )(parts[0, :N_NODES], parts[1, :N_NODES])


@jax.jit
def kernel(embed, adj_sp_norm, edge_index, edge_weight, deg):
  pad = E_PAD - E
  row = jnp.concatenate([edge_index[0], jnp.zeros((pad,), jnp.int32)])
  col = jnp.concatenate([edge_index[1], jnp.zeros((pad,), jnp.int32)])
  w = jnp.concatenate([edge_weight, jnp.zeros((pad,), jnp.float32)])
  row2d = row.reshape(NS * NCHUNK, CHUNK)
  col2d = col.reshape(NS * NCHUNK, CHUNK)
  w2d = w.reshape(NS * NCHUNK, CHUNK)

  embs = [embed]
  a = embed
  for _ in range(3):
    parts = _hop(a, row2d, col2d, w2d)
    a = _combine(parts)
    embs.append(a)
  embs = jnp.stack(embs, axis=1)
  return (embs[: N_NODES // 2], embs[N_NODES // 2:])


# speedup vs baseline: 1.3010x; 1.0017x over previous
import functools

import jax
import jax.numpy as jnp
from jax import lax
from jax.experimental import pallas as pl
from jax.experimental.pallas import tpu as pltpu
from jax.experimental.pallas import tpu_sc as plsc

N_NODES = 10000
D = 128
E = 320000
NC = 2
NS = 16
NW = NC * NS
CHUNK = 128
K0 = 120
K1 = 40
NCHUNK = K0 + K1
E_PAD = NS * NCHUNK * CHUNK
N_PAD = 10240
ROWS_PER_TILE = N_PAD // NS
ZROWS = 128


SDEPTH = 40
NSTAGE0 = K0 // SDEPTH
NSTAGE1 = K1 // SDEPTH


NBUF = 2


def _hop_body(src_hbm, row_hbm, col_hbm, w_hbm, parts_hbm,
              acc_sh, rows0, rows1, ridx_v, cidx_v, w_v,
              gsem0, gsem1, ssem0, ssem1):
  cid = lax.axis_index("c")
  sid = lax.axis_index("s")
  wid = sid * NC + cid

  bufs = (rows0, rows1)
  gsems = (gsem0, gsem1)
  ssems = (ssem0, ssem1)

  def _zero_row(r, _):
    for j in range(D // 16):
      rows0[r, pl.ds(j * 16, 16)] = jnp.zeros((16,), jnp.float32)
    return 0
  lax.fori_loop(0, ZROWS, _zero_row, 0)
  for k in range(ROWS_PER_TILE // ZROWS):
    pltpu.async_copy(rows0, acc_sh.at[pl.ds(sid * ROWS_PER_TILE + k * ZROWS,
                                            ZROWS)], ssem0)
  for k in range(ROWS_PER_TILE // ZROWS):
    pltpu.make_async_copy(rows0, acc_sh.at[pl.ds(sid * ROWS_PER_TILE
                                                 + k * ZROWS, ZROWS)],
                          ssem0).wait()
  plsc.subcore_barrier()

  def _run_stage(cbase):
    pltpu.async_copy(row_hbm.at[pl.ds(cbase, SDEPTH)], ridx_v, ssem1)
    pltpu.async_copy(col_hbm.at[pl.ds(cbase, SDEPTH)], cidx_v, ssem1)
    pltpu.async_copy(w_hbm.at[pl.ds(cbase, SDEPTH)], w_v, ssem1)
    pltpu.make_async_copy(row_hbm.at[pl.ds(cbase, SDEPTH)], ridx_v,
                          ssem1).wait()
    pltpu.make_async_copy(col_hbm.at[pl.ds(cbase, SDEPTH)], cidx_v,
                          ssem1).wait()
    pltpu.make_async_copy(w_hbm.at[pl.ds(cbase, SDEPTH)], w_v, ssem1).wait()

    pltpu.async_copy(src_hbm.at[ridx_v.at[0]], bufs[0], gsems[0])
    pltpu.async_copy(src_hbm.at[ridx_v.at[1]], bufs[1], gsems[1])

    def _pair(k, _):
      for b in range(NBUF):
        lj = NBUF * k + b

        pltpu.make_async_copy(src_hbm.at[ridx_v.at[lj]], bufs[b],
                              gsems[b]).wait()

        @plsc.parallel_loop(0, CHUNK, 1, unroll=4)
        def _scale(e):
          ws = plsc.load_gather(w_v, [jnp.full((16,), lj, jnp.int32),
                                      jnp.full((16,), e, jnp.int32)])
          for j in range(D // 16):
            sl = pl.ds(j * 16, 16)
            bufs[b][e, sl] = bufs[b][e, sl] * ws

        pltpu.sync_copy(bufs[b], acc_sh.at[cidx_v.at[lj]], add=True)

        @pl.when(lj + 2 < SDEPTH)
        def _():
          pltpu.async_copy(src_hbm.at[ridx_v.at[lj + 2]], bufs[b], gsems[b])
      return 0
    lax.fori_loop(0, SDEPTH // NBUF, _pair, 0)

  def _stage(s, _):
    nstage = jnp.where(cid == 0, NSTAGE0, NSTAGE1)

    @pl.when(s < nstage)
    def _():
      cbase = jnp.where(cid == 0, sid * K0 + s * SDEPTH,
                        NS * K0 + sid * K1 + s * SDEPTH)
      _run_stage(pl.multiple_of(cbase, 8))
    return 0
  lax.fori_loop(0, NSTAGE0, _stage, 0)

  plsc.subcore_barrier()

  base = sid * ROWS_PER_TILE
  pltpu.async_copy(acc_sh.at[pl.ds(base, ROWS_PER_TILE)],
                   parts_hbm.at[cid, pl.ds(base, ROWS_PER_TILE)], ssem0)
  pltpu.make_async_copy(acc_sh.at[pl.ds(base, ROWS_PER_TILE)],
                        parts_hbm.at[cid, pl.ds(base, ROWS_PER_TILE)],
                        ssem0).wait()


_hop = pl.kernel(
    _hop_body,
    out_type=jax.ShapeDtypeStruct((NC, N_PAD, D), jnp.float32),
    mesh=plsc.VectorSubcoreMesh(core_axis_name="c", subcore_axis_name="s",
                                num_cores=NC, num_subcores=NS),
    scratch_types=[
        pltpu.VMEM_SHARED((N_PAD, D), jnp.float32),
        pltpu.VMEM((CHUNK, D), jnp.float32),
        pltpu.VMEM((CHUNK, D), jnp.float32),
        pltpu.VMEM((SDEPTH, CHUNK), jnp.int32),
        pltpu.VMEM((SDEPTH, CHUNK), jnp.int32),
        pltpu.VMEM((SDEPTH, CHUNK), jnp.float32),
        pltpu.SemaphoreType.DMA,
        pltpu.SemaphoreType.DMA,
        pltpu.SemaphoreType.DMA,
        pltpu.SemaphoreType.DMA,
    ],
    compiler_params=pltpu.CompilerParams(needs_layout_passes=False),
)


def _add_body(a_ref, b_ref, o_ref):
  o_ref[...] = a_ref[...] + b_ref[...]


_BLK = 2000


def _combine(parts):
  return pl.pallas_call(
      _add_body,
      grid=(N_NODES // _BLK,),
      in_specs=[pl.BlockSpec((_BLK, D), lambda i: (i, 0)),
                pl.BlockSpec((_BLK, D), lambda i: (i, 0))],
      out_specs=pl.BlockSpec((_BLK, D), lambda i: (i, 0)),
      out_shape=jax.ShapeDtypeStruct((N_NODES, D), jnp.float32),
  )(parts[0, :N_NODES], parts[1, :N_NODES])


@jax.jit
def kernel(embed, adj_sp_norm, edge_index, edge_weight, deg):
  pad = E_PAD - E
  row = jnp.concatenate([edge_index[0], jnp.zeros((pad,), jnp.int32)])
  col = jnp.concatenate([edge_index[1], jnp.zeros((pad,), jnp.int32)])
  w = jnp.concatenate([edge_weight, jnp.zeros((pad,), jnp.float32)])
  row2d = row.reshape(NS * NCHUNK, CHUNK)
  col2d = col.reshape(NS * NCHUNK, CHUNK)
  w2d = w.reshape(NS * NCHUNK, CHUNK)

  embs = [embed]
  a = embed
  for _ in range(3):
    parts = _hop(a, row2d, col2d, w2d)
    a = _combine(parts)
    embs.append(a)
  embs = jnp.stack(embs, axis=1)
  return (embs[: N_NODES // 2], embs[N_NODES // 2:])
